# baseline (speedup 1.0000x reference)
import functools

import jax
import jax.numpy as jnp
from jax import lax
from jax.experimental import pallas as pl
from jax.experimental.pallas import tpu as pltpu
from jax.experimental.pallas import tpu_sc as plsc

CHUNK = 200


@functools.lru_cache(maxsize=None)
def _build_gather(total, n_rows, hidden):
    info = plsc.get_sparse_core_info()
    nc, ns = info.num_cores, info.num_subcores
    nw = nc * ns
    per_w = total // nw
    n_chunks = per_w // CHUNK
    assert n_chunks % 4 == 0
    rows_per_tile = n_rows // ns
    mesh = plsc.VectorSubcoreMesh(core_axis_name="c", subcore_axis_name="s")

    @functools.partial(
        pl.kernel,
        mesh=mesh,
        out_type=jax.ShapeDtypeStruct((total, hidden), jnp.float32),
        scratch_types=[
            pltpu.VMEM((per_w,), jnp.int32),
            pltpu.VMEM((CHUNK, hidden), jnp.float32),
            pltpu.VMEM((CHUNK, hidden), jnp.float32),
            pltpu.VMEM((CHUNK, hidden), jnp.float32),
            pltpu.VMEM((CHUNK, hidden), jnp.float32),
            pltpu.VMEM_SHARED((n_rows, hidden), jnp.float32),
            pltpu.SemaphoreType.DMA,
            pltpu.SemaphoreType.DMA,
            pltpu.SemaphoreType.DMA,
            pltpu.SemaphoreType.DMA,
        ],
    )
    def gather_kernel(table_hbm, idx_hbm, out_hbm, idx_v, rows0, rows1,
                      rows2, rows3, table_sp, sem0, sem1, sem2, sem3):
        cid = lax.axis_index("c")
        sid = lax.axis_index("s")
        wid = sid * nc + cid
        base = wid * per_w

        idx_cp = pltpu.make_async_copy(idx_hbm.at[pl.ds(base, per_w)], idx_v,
                                       sem0)
        idx_cp.start()
        stripe = sid * rows_per_tile
        pltpu.sync_copy(table_hbm.at[pl.ds(stripe, rows_per_tile)],
                        table_sp.at[pl.ds(stripe, rows_per_tile)])
        plsc.subcore_barrier()
        idx_cp.wait()

        bufs = (rows0, rows1, rows2, rows3)
        sems = (sem0, sem1, sem2, sem3)
        nb = len(bufs)

        def start_gather(i, buf, sem, from_hbm):
            src = table_hbm if from_hbm else table_sp
            pltpu.async_copy(src.at[idx_v.at[pl.ds(i * CHUNK, CHUNK)]],
                             buf, sem)

        def wait_gather(i, buf, sem, from_hbm):
            src = table_hbm if from_hbm else table_sp
            pltpu.make_async_copy(
                src.at[idx_v.at[pl.ds(i * CHUNK, CHUNK)]], buf, sem
            ).wait()

        def start_out(i, buf, sem):
            pltpu.async_copy(buf, out_hbm.at[pl.ds(base + i * CHUNK, CHUNK)],
                             sem)

        def wait_out(i, buf, sem):
            pltpu.make_async_copy(
                buf, out_hbm.at[pl.ds(base + i * CHUNK, CHUNK)], sem
            ).wait()

        start_gather(0, bufs[0], sems[0], from_hbm=False)

        def ring_body(p, carry):
            for b in range(nb):
                i = nb * p + b
                nxt = (b + 1) % nb
                wait_gather(i, bufs[b], sems[b], from_hbm=(b % 2 == 1))
                start_out(i, bufs[b], sems[b])

                @pl.when(i + 1 < n_chunks)
                def _():
                    @pl.when(i + 1 >= nb)
                    def _():
                        wait_out(i + 1 - nb, bufs[nxt], sems[nxt])

                    start_gather(i + 1, bufs[nxt], sems[nxt],
                                 from_hbm=(nxt % 2 == 1))
            return carry

        lax.fori_loop(0, n_chunks // nb, ring_body, 0)
        for b in range(nb):
            i = n_chunks - nb + b
            wait_out(i, bufs[i % nb], sems[i % nb])

    return gather_kernel


def kernel(top_vecs, position_ids, pos_table):
    del top_vecs
    b, s = position_ids.shape
    idx = position_ids.reshape(-1).astype(jnp.int32)
    out = _build_gather(b * s, pos_table.shape[0], pos_table.shape[1])(
        pos_table, idx)
    return out.reshape(b, s, pos_table.shape[1])


# speedup vs baseline: 1.5396x; 1.5396x over previous
import functools

import jax
import jax.numpy as jnp
from jax import lax
from jax.experimental import pallas as pl
from jax.experimental.pallas import tpu as pltpu
from jax.experimental.pallas import tpu_sc as plsc

CHUNK = 80


@functools.lru_cache(maxsize=None)
def _build_gather(total, n_rows, hidden):
    info = plsc.get_sparse_core_info()
    nc, ns = info.num_cores, info.num_subcores
    nw = nc * ns
    per_w = total // nw
    n_chunks = per_w // CHUNK
    assert n_chunks % 8 == 0
    rows_per_tile = n_rows // ns
    mesh = plsc.VectorSubcoreMesh(core_axis_name="c", subcore_axis_name="s")

    @functools.partial(
        pl.kernel,
        mesh=mesh,
        out_type=jax.ShapeDtypeStruct((total, hidden), jnp.float32),
        scratch_types=[
            pltpu.VMEM((per_w,), jnp.int32),
            pltpu.VMEM((CHUNK, hidden), jnp.float32),
            pltpu.VMEM((CHUNK, hidden), jnp.float32),
            pltpu.VMEM((CHUNK, hidden), jnp.float32),
            pltpu.VMEM((CHUNK, hidden), jnp.float32),
            pltpu.VMEM((CHUNK, hidden), jnp.float32),
            pltpu.VMEM((CHUNK, hidden), jnp.float32),
            pltpu.VMEM((CHUNK, hidden), jnp.float32),
            pltpu.VMEM((CHUNK, hidden), jnp.float32),
            pltpu.VMEM_SHARED((n_rows, hidden), jnp.float32),
            pltpu.SemaphoreType.DMA,
            pltpu.SemaphoreType.DMA,
            pltpu.SemaphoreType.DMA,
            pltpu.SemaphoreType.DMA,
            pltpu.SemaphoreType.DMA,
            pltpu.SemaphoreType.DMA,
            pltpu.SemaphoreType.DMA,
            pltpu.SemaphoreType.DMA,
        ],
    )
    def gather_kernel(table_hbm, idx_hbm, out_hbm, idx_v, rows0, rows1,
                      rows2, rows3, rows4, rows5, rows6, rows7, table_sp,
                      sem0, sem1, sem2, sem3, sem4, sem5, sem6, sem7):
        cid = lax.axis_index("c")
        sid = lax.axis_index("s")
        wid = sid * nc + cid
        base = wid * per_w

        idx_cp = pltpu.make_async_copy(idx_hbm.at[pl.ds(base, per_w)], idx_v,
                                       sem0)
        idx_cp.start()
        stripe = sid * rows_per_tile
        pltpu.sync_copy(table_hbm.at[pl.ds(stripe, rows_per_tile)],
                        table_sp.at[pl.ds(stripe, rows_per_tile)])
        plsc.subcore_barrier()
        idx_cp.wait()

        bufs = (rows0, rows1, rows2, rows3, rows4, rows5,
                rows6, rows7)
        sems = (sem0, sem1, sem2, sem3, sem4, sem5, sem6,
                sem7)
        nb = len(bufs)

        def start_gather(i, buf, sem, from_hbm):
            src = table_hbm if from_hbm else table_sp
            pltpu.async_copy(src.at[idx_v.at[pl.ds(i * CHUNK, CHUNK)]],
                             buf, sem)

        def wait_gather(i, buf, sem, from_hbm):
            src = table_hbm if from_hbm else table_sp
            pltpu.make_async_copy(
                src.at[idx_v.at[pl.ds(i * CHUNK, CHUNK)]], buf, sem
            ).wait()

        def start_out(i, buf, sem):
            pltpu.async_copy(buf, out_hbm.at[pl.ds(base + i * CHUNK, CHUNK)],
                             sem)

        def wait_out(i, buf, sem):
            pltpu.make_async_copy(
                buf, out_hbm.at[pl.ds(base + i * CHUNK, CHUNK)], sem
            ).wait()

        LOOKAHEAD = 4
        for j in range(LOOKAHEAD):
            start_gather(j, bufs[j], sems[j], from_hbm=(j % nb == 7))

        def ring_body(p, carry):
            for b in range(nb):
                i = nb * p + b
                ahead = (b + LOOKAHEAD) % nb
                wait_gather(i, bufs[b], sems[b], from_hbm=(b == 7))
                start_out(i, bufs[b], sems[b])

                @pl.when(i + LOOKAHEAD < n_chunks)
                def _():
                    @pl.when(i + LOOKAHEAD >= nb)
                    def _():
                        wait_out(i + LOOKAHEAD - nb, bufs[ahead], sems[ahead])

                    start_gather(i + LOOKAHEAD, bufs[ahead], sems[ahead],
                                 from_hbm=(ahead == 7))
            return carry

        lax.fori_loop(0, n_chunks // nb, ring_body, 0)
        for b in range(nb):
            i = n_chunks - nb + b
            wait_out(i, bufs[i % nb], sems[i % nb])

    return gather_kernel


def kernel(top_vecs, position_ids, pos_table):
    del top_vecs
    b, s = position_ids.shape
    idx = position_ids.reshape(-1).astype(jnp.int32)
    out = _build_gather(b * s, pos_table.shape[0], pos_table.shape[1])(
        pos_table, idx)
    return out.reshape(b, s, pos_table.shape[1])


# speedup vs baseline: 1.9048x; 1.2372x over previous
import functools

import jax
import jax.numpy as jnp
from jax import lax
from jax.experimental import pallas as pl
from jax.experimental.pallas import tpu as pltpu
from jax.experimental.pallas import tpu_sc as plsc

CHUNK = 200


@functools.lru_cache(maxsize=None)
def _build_gather(total, n_rows, hidden):
    info = plsc.get_sparse_core_info()
    nc, ns = info.num_cores, info.num_subcores
    nw = nc * ns
    per_w = total // nw
    n_chunks = per_w // CHUNK
    assert n_chunks % 4 == 0
    rows_per_tile = n_rows // ns
    mesh = plsc.VectorSubcoreMesh(core_axis_name="c", subcore_axis_name="s")

    @functools.partial(
        pl.kernel,
        mesh=mesh,
        out_type=jax.ShapeDtypeStruct((total, hidden), jnp.float32),
        scratch_types=[
            pltpu.VMEM((per_w,), jnp.int32),
            pltpu.VMEM((CHUNK, hidden), jnp.float32),
            pltpu.VMEM((CHUNK, hidden), jnp.float32),
            pltpu.VMEM((CHUNK, hidden), jnp.float32),
            pltpu.VMEM((CHUNK, hidden), jnp.float32),
            pltpu.VMEM_SHARED((n_rows, hidden), jnp.float32),
            pltpu.SemaphoreType.DMA,
            pltpu.SemaphoreType.DMA,
            pltpu.SemaphoreType.DMA,
            pltpu.SemaphoreType.DMA,
        ],
    )
    def gather_kernel(table_hbm, idx_hbm, out_hbm, idx_v, rows0, rows1,
                      rows2, rows3, table_sp, sem0, sem1, sem2, sem3):
        cid = lax.axis_index("c")
        sid = lax.axis_index("s")
        wid = sid * nc + cid
        base = wid * per_w

        idx_cp = pltpu.make_async_copy(idx_hbm.at[pl.ds(base, per_w)], idx_v,
                                       sem0)
        idx_cp.start()
        stripe = sid * rows_per_tile
        pltpu.sync_copy(table_hbm.at[pl.ds(stripe, rows_per_tile)],
                        table_sp.at[pl.ds(stripe, rows_per_tile)])
        plsc.subcore_barrier()
        idx_cp.wait()

        bufs = (rows0, rows1, rows2, rows3)
        sems = (sem0, sem1, sem2, sem3)
        nb = len(bufs)

        def start_gather(i, buf, sem):
            pltpu.async_copy(table_sp.at[idx_v.at[pl.ds(i * CHUNK, CHUNK)]],
                             buf, sem)

        def wait_gather(i, buf, sem):
            pltpu.make_async_copy(
                table_sp.at[idx_v.at[pl.ds(i * CHUNK, CHUNK)]], buf, sem
            ).wait()

        def start_out(i, buf, sem):
            pltpu.async_copy(buf, out_hbm.at[pl.ds(base + i * CHUNK, CHUNK)],
                             sem)

        def wait_out(i, buf, sem):
            pltpu.make_async_copy(
                buf, out_hbm.at[pl.ds(base + i * CHUNK, CHUNK)], sem
            ).wait()

        start_gather(0, bufs[0], sems[0])

        def ring_body(p, carry):
            for b in range(nb):
                i = nb * p + b
                nxt = (b + 1) % nb
                wait_gather(i, bufs[b], sems[b])
                start_out(i, bufs[b], sems[b])

                @pl.when(i + 1 < n_chunks)
                def _():
                    @pl.when(i + 1 >= nb)
                    def _():
                        wait_out(i + 1 - nb, bufs[nxt], sems[nxt])

                    start_gather(i + 1, bufs[nxt], sems[nxt])
            return carry

        lax.fori_loop(0, n_chunks // nb, ring_body, 0)
        for b in range(nb):
            i = n_chunks - nb + b
            wait_out(i, bufs[i % nb], sems[i % nb])

    return gather_kernel


def kernel(top_vecs, position_ids, pos_table):
    del top_vecs
    b, s = position_ids.shape
    idx = position_ids.reshape(-1).astype(jnp.int32)
    out = _build_gather(b * s, pos_table.shape[0], pos_table.shape[1])(
        pos_table, idx)
    return out.reshape(b, s, pos_table.shape[1])
